# Initial kernel scaffold; baseline (speedup 1.0000x reference)
import jax
import jax.numpy as jnp
from jax.experimental import pallas as pl


def kernel(x, indices):
    raise NotImplementedError("write your pallas kernel here")


# speedup vs baseline: 7.4031x; 7.4031x over previous
import functools

import jax
import jax.numpy as jnp
from jax import lax
from jax.experimental import pallas as pl
from jax.experimental.pallas import tpu as pltpu
from jax.experimental.pallas import tpu_sc as plsc

_N, _C, _H, _W = 64, 384, 24, 24
_HW = _H * _W
_RPB = 4


def _tc_copy_min_body(x_ref, y_ref, mb_ref):
    i = pl.program_id(0)

    @pl.when(i == 0)
    def _init():
        mb_ref[...] = jnp.full((1, 128), jnp.inf, jnp.float32)

    v = x_ref[...]
    y_ref[...] = v
    mb_ref[...] = jnp.minimum(mb_ref[...], jnp.min(v))


_tc_pass = pl.pallas_call(
    _tc_copy_min_body,
    grid=(_N // _RPB,),
    in_specs=[pl.BlockSpec((_RPB, _C, _HW), lambda i: (i, 0, 0))],
    out_specs=[
        pl.BlockSpec((_RPB, _C, _HW), lambda i: (i, 0, 0)),
        pl.BlockSpec((1, 128), lambda i: (0, 0)),
    ],
    out_shape=[
        jax.ShapeDtypeStruct((_N, _C, _HW), jnp.float32),
        jax.ShapeDtypeStruct((1, 128), jnp.float32),
    ],
)


_sc_scratch = [
    pltpu.VMEM((_N, _HW), jnp.float32),
    pltpu.VMEM((_N,), jnp.int32),
    pltpu.VMEM((128,), jnp.float32),
    pltpu.SemaphoreType.DMA,
]


def _sc_scatter_body(y_hbm, idx_hbm, mb_hbm, vals_v, idx_v, mb_v, sem):
    cid = lax.axis_index("c")
    sid = lax.axis_index("s")

    @pl.when(jnp.logical_and(cid == 0, sid == 0))
    def _():
        pltpu.sync_copy(idx_hbm, idx_v)
        pltpu.sync_copy(mb_hbm, mb_v)

        acc = mb_v[pl.ds(0, 16)]
        for k in range(1, 8):
            acc = jnp.minimum(acc, mb_v[pl.ds(16 * k, 16)])
        m0 = jnp.min(acc)

        def body(i, m):
            val = jnp.where(m == 0.0, jnp.float32(0.0), m - jnp.float32(1e7))
            vv = jnp.full((16,), val, jnp.float32)
            for k in range(_HW // 16):
                vals_v[i, pl.ds(16 * k, 16)] = vv
            return val

        lax.fori_loop(0, _N, body, m0)

        chunks = [idx_v[pl.ds(16 * a, 16)] for a in range(_N // 16)]
        copies = []
        for i in range(_N):
            ch = chunks[i // 16][i % 16]
            row = jnp.int32(i * _C) + ch
            copies.append(pltpu.async_copy(vals_v.at[i], y_hbm.at[row], sem))
        for cp in copies:
            cp.wait()


@functools.cache
def _get_sc_scatter():
    mesh = plsc.VectorSubcoreMesh(core_axis_name="c", subcore_axis_name="s")
    return pl.kernel(
        _sc_scatter_body,
        out_type=(),
        mesh=mesh,
        scratch_types=_sc_scratch,
        compiler_params=pltpu.CompilerParams(needs_layout_passes=False),
    )


def kernel(x, indices):
    y, mb = _tc_pass(x.reshape(_N, _C, _HW))
    y_ref = jax.new_ref(y.reshape(_N * _C, _HW))
    _get_sc_scatter()(y_ref, indices, mb.reshape(128))
    return jax.freeze(y_ref).reshape(_N, _C, _H, _W)
